# baseline (speedup 1.0000x reference)
import functools

import jax
import jax.numpy as jnp
from jax import lax
from jax.experimental import pallas as pl
from jax.experimental.pallas import tpu as pltpu
from jax.experimental.pallas import tpu_sc as plsc

B = 8
N = 32768
K = 2048
CAP = 6144

_SC_CORES = 2
_SC_SUBCORES = 16
_M = 0x7FFFFFFF
_SIGN = -2147483648
_NV = N // 16


def _skey(v):
    bb = lax.bitcast_convert_type(v, jnp.int32)
    keyv = jnp.where(bb < 0, bb ^ _M, bb)
    return keyv ^ _M


def _unskey(sk):
    return jnp.where(sk < 0, sk, sk ^ _M)


@functools.lru_cache(maxsize=None)
def _make_sc_topk():
    mesh = plsc.VectorSubcoreMesh(core_axis_name="c", subcore_axis_name="s")
    n_gchunk = 2 * K // 128
    kbuf = K + CAP + 32

    @functools.partial(
        pl.kernel,
        mesh=mesh,
        compiler_params=pltpu.CompilerParams(needs_layout_passes=False),
        out_type=(
            jax.ShapeDtypeStruct((B, K), jnp.float32),
            jax.ShapeDtypeStruct((B, 2 * K), jnp.float32),
        ),
        scratch_types=[
            pltpu.VMEM((N,), jnp.float32),
            pltpu.VMEM((kbuf,), jnp.int32),
            pltpu.VMEM((kbuf,), jnp.int32),
            pltpu.VMEM((kbuf,), jnp.int32),
            pltpu.VMEM((kbuf,), jnp.int32),
            pltpu.VMEM((2064,), jnp.int32),
            pltpu.VMEM((2064,), jnp.int32),
            pltpu.VMEM((16,), jnp.int32),
            pltpu.VMEM((n_gchunk, 128), jnp.int32),
            pltpu.VMEM((2 * K,), jnp.float32),
            pltpu.VMEM((K,), jnp.float32),
            pltpu.SemaphoreType.DMA,
        ],
    )
    def topk_k(
        scores_hbm, kflat_hbm, oscore_hbm, okpts_hbm,
        sv, kb0, ib0, kb1, ib1, hist, base, spill, gidx, gbuf, sbuf, sem,
    ):
        wid = lax.axis_index("s") * _SC_CORES + lax.axis_index("c")

        @pl.when(wid < B)
        def _():
            pltpu.sync_copy(scores_hbm.at[wid], sv)
            lanes = lax.broadcasted_iota(jnp.int32, (16,), 0)
            zeros16 = jnp.zeros((16,), jnp.int32)

            def run_level(sh, msk, nbins, lo, hi, below, first):
                for j in range(2064 // 16):
                    hist[pl.ds(j * 16, 16)] = zeros16

                def hbody(i, carry):
                    v = sv[pl.ds(i * 16, 16)]
                    sk = _skey(v)
                    d = (sk >> sh) & msk
                    if not first:
                        ssk = sk ^ _SIGN
                        cand = (ssk >= lo) & (ssk <= hi)
                        d = jnp.where(cand, d, nbins)
                    a, last = plsc.scan_count(d)
                    plsc.addupdate_scatter(hist, [d], a, mask=last)
                    return carry

                lax.fori_loop(0, _NV, hbody, jnp.int32(0))

                local_rank = K - below

                def pbody(j, carry):
                    run, nlt = carry
                    hv = hist[pl.ds(j * 16, 16)]
                    c = plsc.cumsum(hv) + run
                    base[pl.ds(j * 16, 16)] = c - hv
                    nlt = nlt + plsc.all_reduce_population_count(
                        c < local_rank
                    )[0]
                    return run + jnp.sum(hv), nlt

                _, bstar = lax.fori_loop(
                    0, nbins // 16, pbody, (jnp.int32(0), jnp.int32(0))
                )
                bvec = jnp.broadcast_to(bstar, (16,))
                below_here = plsc.load_gather(base, [bvec])[0]
                cnt = plsc.load_gather(hist, [bvec])[0]
                return bstar, below + below_here, cnt

            b0, below, cnt = run_level(21, 0x7FF, 2048, 0, 0, jnp.int32(0), True)
            ub = b0 << 21
            span = 1 << 21

            def lvl(sh, msk, nbins, ub, span, below, cnt):
                lo = ub ^ _SIGN
                hi = (ub + span - 1) ^ _SIGN
                bn, below2, cnt2 = run_level(sh, msk, nbins, lo, hi, below, False)
                return ub | (bn << sh), 1 << sh, below2, cnt2

            st = (ub, span, below, cnt)

            def refine1(st):
                return lvl(10, 0x7FF, 2048, st[0], st[1], st[2], st[3])

            def refine2(st):
                return lvl(0, 0x3FF, 1024, st[0], st[1], st[2], st[3])

            st = lax.cond(st[3] > CAP, refine1, lambda s: s, st)
            st = lax.cond(st[3] > CAP, refine2, lambda s: s, st)
            ub, span, below, cnt = st

            lo_cut = jnp.broadcast_to(ub ^ _SIGN, (16,))
            hi_cut = jnp.broadcast_to((ub + span - 1) ^ _SIGN, (16,))
            cap_total = jnp.where(cnt <= CAP, below + cnt, jnp.int32(K))

            def cbody(i, carry):
                w_off, c_off = carry
                v = sv[pl.ds(i * 16, 16)]
                sk = _skey(v)
                ssk = sk ^ _SIGN
                iv = lanes + i * 16
                win = ssk < lo_cut
                cand = (ssk >= lo_cut) & (ssk <= hi_cut)
                nw = plsc.all_reduce_population_count(win)[0]
                plsc.store_compressed(kb0.at[pl.ds(w_off, 16)], sk, mask=win)
                plsc.store_compressed(ib0.at[pl.ds(w_off, 16)], iv, mask=win)
                cc = plsc.cumsum(cand.astype(jnp.int32))
                keep = cand & ((c_off + cc) <= cap_total)
                nk = plsc.all_reduce_population_count(keep)[0]
                plsc.store_compressed(kb0.at[pl.ds(c_off, 16)], sk, mask=keep)
                plsc.store_compressed(ib0.at[pl.ds(c_off, 16)], iv, mask=keep)
                return w_off + nw, c_off + nk

            lax.fori_loop(0, _NV, cbody, (jnp.int32(0), below))

            kb0[pl.ds(cap_total, 16)] = jnp.full((16,), -1, jnp.int32)
            ib0[pl.ds(cap_total, 16)] = jnp.full((16,), N, jnp.int32)
            nv_sort = (cap_total + 15) >> 4

            bufs = [(kb0, ib0, kb1, ib1), (kb1, ib1, kb0, ib0)]
            for p in range(4):
                src_k, src_i, dst_k, dst_i = bufs[p % 2]
                shift = 8 * p
                for j in range(16):
                    hist[pl.ds(j * 16, 16)] = zeros16

                def hbody2(i, carry, src_k=src_k, shift=shift):
                    kv = src_k[pl.ds(i * 16, 16)]
                    d = (kv >> shift) & 255
                    a, last = plsc.scan_count(d)
                    plsc.addupdate_scatter(hist, [d], a, mask=last)
                    return carry

                lax.fori_loop(0, nv_sort, hbody2, jnp.int32(0))

                def pbody2(j, run):
                    hv = hist[pl.ds(j * 16, 16)]
                    c = plsc.cumsum(hv)
                    base[pl.ds(j * 16, 16)] = (run + c) - hv
                    return run + jnp.sum(hv)

                lax.fori_loop(0, 16, pbody2, jnp.int32(0))

                def mbody2(i, carry, src_k=src_k, src_i=src_i,
                           dst_k=dst_k, dst_i=dst_i, shift=shift):
                    kv = src_k[pl.ds(i * 16, 16)]
                    ix = src_i[pl.ds(i * 16, 16)]
                    d = (kv >> shift) & 255
                    a, last = plsc.scan_count(d)
                    bs = plsc.load_gather(base, [d])
                    pos = bs + a - 1
                    plsc.store_scatter(dst_k, [pos], kv)
                    plsc.store_scatter(dst_i, [pos], ix)
                    plsc.addupdate_scatter(base, [d], a, mask=last)
                    return carry

                lax.fori_loop(0, nv_sort, mbody2, jnp.int32(0))


            rbase2 = wid * (2 * N)

            def ebody(i, carry):
                sk = kb0[pl.ds(i * 16, 16)]
                sbuf[pl.ds(i * 16, 16)] = lax.bitcast_convert_type(
                    _unskey(sk), jnp.float32
                )
                ix = ib0[pl.ds(i * 16, 16)]
                e0 = rbase2 + ix * 2
                p0 = (lanes + i * 16) * 2
                plsc.store_scatter(gidx, [p0 >> 7, p0 & 127], e0)
                plsc.store_scatter(gidx, [(p0 + 1) >> 7, (p0 + 1) & 127], e0 + 1)
                return carry

            lax.fori_loop(0, K // 16, ebody, jnp.int32(0))
            pltpu.sync_copy(sbuf, oscore_hbm.at[wid])

            copies = [
                pltpu.async_copy(
                    kflat_hbm.at[gidx.at[j]], gbuf.at[pl.ds(j * 128, 128)], sem
                )
                for j in range(n_gchunk)
            ]
            for cpy in copies:
                cpy.wait()
            pltpu.sync_copy(gbuf, okpts_hbm.at[wid])

    return topk_k


def kernel(keypoints, scores):
    top_scores, kpts_flat = _make_sc_topk()(scores, keypoints.reshape(2 * B * N))
    return (kpts_flat.reshape(B, K, 2), top_scores)


# speedup vs baseline: 1.0139x; 1.0139x over previous
import functools

import jax
import jax.numpy as jnp
from jax import lax
from jax.experimental import pallas as pl
from jax.experimental.pallas import tpu as pltpu
from jax.experimental.pallas import tpu_sc as plsc

B = 8
N = 32768
K = 2048
CAP = 6144

_SC_CORES = 2
_SC_SUBCORES = 16
_M = 0x7FFFFFFF
_SIGN = -2147483648
_NV = N // 16


def _skey(v):
    bb = lax.bitcast_convert_type(v, jnp.int32)
    keyv = jnp.where(bb < 0, bb ^ _M, bb)
    return keyv ^ _M


def _unskey(sk):
    return jnp.where(sk < 0, sk, sk ^ _M)


@functools.lru_cache(maxsize=None)
def _make_sc_topk():
    mesh = plsc.VectorSubcoreMesh(core_axis_name="c", subcore_axis_name="s")
    n_gchunk = 2 * K // 128
    kbuf = K + CAP + 96

    @functools.partial(
        pl.kernel,
        mesh=mesh,
        compiler_params=pltpu.CompilerParams(needs_layout_passes=False),
        out_type=(
            jax.ShapeDtypeStruct((B, K), jnp.float32),
            jax.ShapeDtypeStruct((B, 2 * K), jnp.float32),
        ),
        scratch_types=[
            pltpu.VMEM((N,), jnp.float32),
            pltpu.VMEM((kbuf,), jnp.int32),
            pltpu.VMEM((kbuf,), jnp.int32),
            pltpu.VMEM((kbuf,), jnp.int32),
            pltpu.VMEM((kbuf,), jnp.int32),
            pltpu.VMEM((2064,), jnp.int32),
            pltpu.VMEM((2064,), jnp.int32),
            pltpu.VMEM((16,), jnp.int32),
            pltpu.VMEM((n_gchunk, 128), jnp.int32),
            pltpu.VMEM((2 * K,), jnp.float32),
            pltpu.VMEM((K,), jnp.float32),
            pltpu.SemaphoreType.DMA,
        ],
    )
    def topk_k(
        scores_hbm, kflat_hbm, oscore_hbm, okpts_hbm,
        sv, kb0, ib0, kb1, ib1, hist, base, spill, gidx, gbuf, sbuf, sem,
    ):
        wid = lax.axis_index("s") * _SC_CORES + lax.axis_index("c")

        @pl.when(wid < B)
        def _():
            pltpu.sync_copy(scores_hbm.at[wid], sv)
            lanes = lax.broadcasted_iota(jnp.int32, (16,), 0)
            zeros16 = jnp.zeros((16,), jnp.int32)

            def run_level(sh, msk, nbins, lo, hi, below, first):
                for j in range(2064 // 16):
                    hist[pl.ds(j * 16, 16)] = zeros16

                def hbody(i, carry):
                    for u in range(8):
                        v = sv[pl.ds((i * 8 + u) * 16, 16)]
                        sk = _skey(v)
                        d = (sk >> sh) & msk
                        if not first:
                            ssk = sk ^ _SIGN
                            cand = (ssk >= lo) & (ssk <= hi)
                            d = jnp.where(cand, d, nbins)
                        a, last = plsc.scan_count(d)
                        plsc.addupdate_scatter(hist, [d], a, mask=last)
                    return carry

                lax.fori_loop(0, _NV // 8, hbody, jnp.int32(0))

                local_rank = K - below

                def pbody(j, carry):
                    run, nlt = carry
                    hv = hist[pl.ds(j * 16, 16)]
                    c = plsc.cumsum(hv) + run
                    base[pl.ds(j * 16, 16)] = c - hv
                    nlt = nlt + plsc.all_reduce_population_count(
                        c < local_rank
                    )[0]
                    return run + jnp.sum(hv), nlt

                _, bstar = lax.fori_loop(
                    0, nbins // 16, pbody, (jnp.int32(0), jnp.int32(0))
                )
                bvec = jnp.broadcast_to(bstar, (16,))
                below_here = plsc.load_gather(base, [bvec])[0]
                cnt = plsc.load_gather(hist, [bvec])[0]
                return bstar, below + below_here, cnt

            b0, below, cnt = run_level(21, 0x7FF, 2048, 0, 0, jnp.int32(0), True)
            ub = b0 << 21
            span = 1 << 21

            def lvl(sh, msk, nbins, ub, span, below, cnt):
                lo = ub ^ _SIGN
                hi = (ub + span - 1) ^ _SIGN
                bn, below2, cnt2 = run_level(sh, msk, nbins, lo, hi, below, False)
                return ub | (bn << sh), 1 << sh, below2, cnt2

            st = (ub, span, below, cnt)

            def refine1(st):
                return lvl(10, 0x7FF, 2048, st[0], st[1], st[2], st[3])

            def refine2(st):
                return lvl(0, 0x3FF, 1024, st[0], st[1], st[2], st[3])

            st = lax.cond(st[3] > CAP, refine1, lambda s: s, st)
            st = lax.cond(st[3] > CAP, refine2, lambda s: s, st)
            ub, span, below, cnt = st

            lo_cut = jnp.broadcast_to(ub ^ _SIGN, (16,))
            hi_cut = jnp.broadcast_to((ub + span - 1) ^ _SIGN, (16,))
            cap_total = jnp.where(cnt <= CAP, below + cnt, jnp.int32(K))

            def cbody(i, carry):
                w_off, c_off = carry
                for u in range(4):
                    ii = i * 4 + u
                    v = sv[pl.ds(ii * 16, 16)]
                    sk = _skey(v)
                    ssk = sk ^ _SIGN
                    iv = lanes + ii * 16
                    win = ssk < lo_cut
                    cand = (ssk >= lo_cut) & (ssk <= hi_cut)
                    nw = plsc.all_reduce_population_count(win)[0]
                    plsc.store_compressed(kb0.at[pl.ds(w_off, 16)], sk, mask=win)
                    plsc.store_compressed(ib0.at[pl.ds(w_off, 16)], iv, mask=win)
                    cc = plsc.cumsum(cand.astype(jnp.int32))
                    keep = cand & ((c_off + cc) <= cap_total)
                    nk = plsc.all_reduce_population_count(keep)[0]
                    plsc.store_compressed(kb0.at[pl.ds(c_off, 16)], sk, mask=keep)
                    plsc.store_compressed(ib0.at[pl.ds(c_off, 16)], iv, mask=keep)
                    w_off = w_off + nw
                    c_off = c_off + nk
                return w_off, c_off

            lax.fori_loop(0, _NV // 4, cbody, (jnp.int32(0), below))

            for u in range(4):
                kb0[pl.ds(cap_total + u * 16, 16)] = jnp.full((16,), -1, jnp.int32)
                ib0[pl.ds(cap_total + u * 16, 16)] = jnp.full((16,), N, jnp.int32)
            nv_sort = (cap_total + 63) >> 6

            bufs = [(kb0, ib0, kb1, ib1), (kb1, ib1, kb0, ib0)]
            for p in range(4):
                src_k, src_i, dst_k, dst_i = bufs[p % 2]
                shift = 8 * p
                for j in range(16):
                    hist[pl.ds(j * 16, 16)] = zeros16

                def hbody2(i, carry, src_k=src_k, shift=shift):
                    for u in range(4):
                        kv = src_k[pl.ds((i * 4 + u) * 16, 16)]
                        d = (kv >> shift) & 255
                        a, last = plsc.scan_count(d)
                        plsc.addupdate_scatter(hist, [d], a, mask=last)
                    return carry

                lax.fori_loop(0, nv_sort, hbody2, jnp.int32(0))

                def pbody2(j, run):
                    hv = hist[pl.ds(j * 16, 16)]
                    c = plsc.cumsum(hv)
                    base[pl.ds(j * 16, 16)] = (run + c) - hv
                    return run + jnp.sum(hv)

                lax.fori_loop(0, 16, pbody2, jnp.int32(0))

                def mbody2(i, carry, src_k=src_k, src_i=src_i,
                           dst_k=dst_k, dst_i=dst_i, shift=shift):
                    for u in range(4):
                        ii = i * 4 + u
                        kv = src_k[pl.ds(ii * 16, 16)]
                        ix = src_i[pl.ds(ii * 16, 16)]
                        d = (kv >> shift) & 255
                        a, last = plsc.scan_count(d)
                        bs = plsc.load_gather(base, [d])
                        pos = bs + a - 1
                        plsc.store_scatter(dst_k, [pos], kv)
                        plsc.store_scatter(dst_i, [pos], ix)
                        plsc.addupdate_scatter(base, [d], a, mask=last)
                    return carry

                lax.fori_loop(0, nv_sort, mbody2, jnp.int32(0))


            rbase2 = wid * (2 * N)

            def ebody(i, carry):
                for u in range(4):
                    ii = i * 4 + u
                    sk = kb0[pl.ds(ii * 16, 16)]
                    sbuf[pl.ds(ii * 16, 16)] = lax.bitcast_convert_type(
                        _unskey(sk), jnp.float32
                    )
                    ix = ib0[pl.ds(ii * 16, 16)]
                    e0 = rbase2 + ix * 2
                    p0 = (lanes + ii * 16) * 2
                    plsc.store_scatter(gidx, [p0 >> 7, p0 & 127], e0)
                    plsc.store_scatter(
                        gidx, [(p0 + 1) >> 7, (p0 + 1) & 127], e0 + 1
                    )
                return carry

            lax.fori_loop(0, K // 16 // 4, ebody, jnp.int32(0))
            pltpu.sync_copy(sbuf, oscore_hbm.at[wid])

            copies = [
                pltpu.async_copy(
                    kflat_hbm.at[gidx.at[j]], gbuf.at[pl.ds(j * 128, 128)], sem
                )
                for j in range(n_gchunk)
            ]
            for cpy in copies:
                cpy.wait()
            pltpu.sync_copy(gbuf, okpts_hbm.at[wid])

    return topk_k


def kernel(keypoints, scores):
    top_scores, kpts_flat = _make_sc_topk()(scores, keypoints.reshape(2 * B * N))
    return (kpts_flat.reshape(B, K, 2), top_scores)


# speedup vs baseline: 1.1517x; 1.1360x over previous
import functools

import jax
import jax.numpy as jnp
from jax import lax
from jax.experimental import pallas as pl
from jax.experimental.pallas import tpu as pltpu
from jax.experimental.pallas import tpu_sc as plsc

B = 8
N = 32768
K = 2048
CAP = 6144

_SC_CORES = 2
_SC_SUBCORES = 16
_M = 0x7FFFFFFF
_SIGN = -2147483648
_NV = N // 16


def _skey(v):
    bb = lax.bitcast_convert_type(v, jnp.int32)
    keyv = jnp.where(bb < 0, bb ^ _M, bb)
    return keyv ^ _M


def _unskey(sk):
    return jnp.where(sk < 0, sk, sk ^ _M)


@functools.lru_cache(maxsize=None)
def _make_sc_topk():
    mesh = plsc.VectorSubcoreMesh(core_axis_name="c", subcore_axis_name="s")
    n_gchunk = 2 * K // 128
    kbuf = K + CAP + 96

    @functools.partial(
        pl.kernel,
        mesh=mesh,
        compiler_params=pltpu.CompilerParams(needs_layout_passes=False),
        out_type=(
            jax.ShapeDtypeStruct((B, K), jnp.float32),
            jax.ShapeDtypeStruct((B, 2 * K), jnp.float32),
        ),
        scratch_types=[
            pltpu.VMEM((N,), jnp.float32),
            pltpu.VMEM((kbuf,), jnp.int32),
            pltpu.VMEM((kbuf,), jnp.int32),
            pltpu.VMEM((kbuf,), jnp.int32),
            pltpu.VMEM((kbuf,), jnp.int32),
            pltpu.VMEM((2064,), jnp.int32),
            pltpu.VMEM((2064,), jnp.int32),
            pltpu.VMEM((16,), jnp.int32),
            pltpu.VMEM((n_gchunk, 128), jnp.int32),
            pltpu.VMEM((2 * K,), jnp.float32),
            pltpu.VMEM((K,), jnp.float32),
            pltpu.SemaphoreType.DMA,
        ],
    )
    def topk_k(
        scores_hbm, kflat_hbm, oscore_hbm, okpts_hbm,
        sv, kb0, ib0, kb1, ib1, hist, base, spill, gidx, gbuf, sbuf, sem,
    ):
        wid = lax.axis_index("s") * _SC_CORES + lax.axis_index("c")

        @pl.when(wid < B)
        def _():
            pltpu.sync_copy(scores_hbm.at[wid], sv)
            lanes = lax.broadcasted_iota(jnp.int32, (16,), 0)
            zeros16 = jnp.zeros((16,), jnp.int32)

            def run_level(sh, msk, nbins, lo, hi, below, first):
                for j in range(2064 // 16):
                    hist[pl.ds(j * 16, 16)] = zeros16

                ones16 = jnp.full((16,), 1, jnp.int32)

                def hbody(i, carry):
                    for u in range(8):
                        v = sv[pl.ds((i * 8 + u) * 16, 16)]
                        sk = _skey(v)
                        d = (sk >> sh) & msk
                        if not first:
                            ssk = sk ^ _SIGN
                            cand = (ssk >= lo) & (ssk <= hi)
                            d = jnp.where(cand, d, nbins)
                        plsc.addupdate_scatter(hist, [d], ones16)
                    return carry

                lax.fori_loop(0, _NV // 8, hbody, jnp.int32(0))

                local_rank = K - below

                def pbody(j, carry):
                    run, nlt = carry
                    hv = hist[pl.ds(j * 16, 16)]
                    c = plsc.cumsum(hv) + run
                    base[pl.ds(j * 16, 16)] = c - hv
                    nlt = nlt + plsc.all_reduce_population_count(
                        c < local_rank
                    )[0]
                    return run + jnp.sum(hv), nlt

                _, bstar = lax.fori_loop(
                    0, nbins // 16, pbody, (jnp.int32(0), jnp.int32(0))
                )
                bvec = jnp.broadcast_to(bstar, (16,))
                below_here = plsc.load_gather(base, [bvec])[0]
                cnt = plsc.load_gather(hist, [bvec])[0]
                return bstar, below + below_here, cnt

            b0, below, cnt = run_level(21, 0x7FF, 2048, 0, 0, jnp.int32(0), True)
            ub = b0 << 21
            span = 1 << 21

            def lvl(sh, msk, nbins, ub, span, below, cnt):
                lo = ub ^ _SIGN
                hi = (ub + span - 1) ^ _SIGN
                bn, below2, cnt2 = run_level(sh, msk, nbins, lo, hi, below, False)
                return ub | (bn << sh), 1 << sh, below2, cnt2

            st = (ub, span, below, cnt)

            def refine1(st):
                return lvl(10, 0x7FF, 2048, st[0], st[1], st[2], st[3])

            def refine2(st):
                return lvl(0, 0x3FF, 1024, st[0], st[1], st[2], st[3])

            st = lax.cond(st[3] > CAP, refine1, lambda s: s, st)
            st = lax.cond(st[3] > CAP, refine2, lambda s: s, st)
            ub, span, below, cnt = st

            lo_cut = jnp.broadcast_to(ub ^ _SIGN, (16,))
            hi_cut = jnp.broadcast_to((ub + span - 1) ^ _SIGN, (16,))
            cap_total = jnp.where(cnt <= CAP, below + cnt, jnp.int32(K))

            def cbody(i, carry):
                w_off, c_off = carry
                for u in range(4):
                    ii = i * 4 + u
                    v = sv[pl.ds(ii * 16, 16)]
                    sk = _skey(v)
                    ssk = sk ^ _SIGN
                    iv = lanes + ii * 16
                    win = ssk < lo_cut
                    cand = (ssk >= lo_cut) & (ssk <= hi_cut)
                    nw = plsc.all_reduce_population_count(win)[0]
                    plsc.store_compressed(kb0.at[pl.ds(w_off, 16)], sk, mask=win)
                    plsc.store_compressed(ib0.at[pl.ds(w_off, 16)], iv, mask=win)
                    nk = plsc.all_reduce_population_count(cand)[0]
                    plsc.store_compressed(kb0.at[pl.ds(c_off, 16)], sk, mask=cand)
                    plsc.store_compressed(ib0.at[pl.ds(c_off, 16)], iv, mask=cand)
                    w_off = w_off + nw
                    c_off = jnp.minimum(c_off + nk, cap_total)
                return w_off, c_off

            lax.fori_loop(0, _NV // 4, cbody, (jnp.int32(0), below))

            for u in range(4):
                kb0[pl.ds(cap_total + u * 16, 16)] = jnp.full((16,), -1, jnp.int32)
                ib0[pl.ds(cap_total + u * 16, 16)] = jnp.full((16,), N, jnp.int32)
            nv_sort = (cap_total + 63) >> 6

            bufs = [(kb0, ib0, kb1, ib1), (kb1, ib1, kb0, ib0)]
            for p in range(4):
                src_k, src_i, dst_k, dst_i = bufs[p % 2]
                shift = 8 * p
                for j in range(16):
                    hist[pl.ds(j * 16, 16)] = zeros16

                ones16 = jnp.full((16,), 1, jnp.int32)

                def hbody2(i, carry, src_k=src_k, shift=shift):
                    for u in range(4):
                        kv = src_k[pl.ds((i * 4 + u) * 16, 16)]
                        d = (kv >> shift) & 255
                        plsc.addupdate_scatter(hist, [d], ones16)
                    return carry

                lax.fori_loop(0, nv_sort, hbody2, jnp.int32(0))

                def pbody2(j, run):
                    hv = hist[pl.ds(j * 16, 16)]
                    c = plsc.cumsum(hv)
                    base[pl.ds(j * 16, 16)] = (run + c) - hv
                    return run + jnp.sum(hv)

                lax.fori_loop(0, 16, pbody2, jnp.int32(0))

                def mbody2(i, carry, src_k=src_k, src_i=src_i,
                           dst_k=dst_k, dst_i=dst_i, shift=shift):
                    for u in range(4):
                        ii = i * 4 + u
                        kv = src_k[pl.ds(ii * 16, 16)]
                        ix = src_i[pl.ds(ii * 16, 16)]
                        d = (kv >> shift) & 255
                        a, last = plsc.scan_count(d)
                        bs = plsc.load_gather(base, [d])
                        pos = bs + a - 1
                        plsc.store_scatter(dst_k, [pos], kv)
                        plsc.store_scatter(dst_i, [pos], ix)
                        plsc.addupdate_scatter(base, [d], a, mask=last)
                    return carry

                lax.fori_loop(0, nv_sort, mbody2, jnp.int32(0))


            rbase2 = wid * (2 * N)

            def ebody(i, carry):
                for u in range(4):
                    ii = i * 4 + u
                    sk = kb0[pl.ds(ii * 16, 16)]
                    sbuf[pl.ds(ii * 16, 16)] = lax.bitcast_convert_type(
                        _unskey(sk), jnp.float32
                    )
                    ix = ib0[pl.ds(ii * 16, 16)]
                    e0 = rbase2 + ix * 2
                    p0 = (lanes + ii * 16) * 2
                    plsc.store_scatter(gidx, [p0 >> 7, p0 & 127], e0)
                    plsc.store_scatter(
                        gidx, [(p0 + 1) >> 7, (p0 + 1) & 127], e0 + 1
                    )
                return carry

            lax.fori_loop(0, K // 16 // 4, ebody, jnp.int32(0))
            pltpu.sync_copy(sbuf, oscore_hbm.at[wid])

            copies = [
                pltpu.async_copy(
                    kflat_hbm.at[gidx.at[j]], gbuf.at[pl.ds(j * 128, 128)], sem
                )
                for j in range(n_gchunk)
            ]
            for cpy in copies:
                cpy.wait()
            pltpu.sync_copy(gbuf, okpts_hbm.at[wid])

    return topk_k


def kernel(keypoints, scores):
    top_scores, kpts_flat = _make_sc_topk()(scores, keypoints.reshape(2 * B * N))
    return (kpts_flat.reshape(B, K, 2), top_scores)


# speedup vs baseline: 1.2377x; 1.0746x over previous
import functools

import jax
import jax.numpy as jnp
from jax import lax
from jax.experimental import pallas as pl
from jax.experimental.pallas import tpu as pltpu
from jax.experimental.pallas import tpu_sc as plsc

B = 8
N = 32768
K = 2048
CAP = 6144

_SC_CORES = 2
_SC_SUBCORES = 16
_M = 0x7FFFFFFF
_SIGN = -2147483648
_NV = N // 16


def _skey(v):
    bb = lax.bitcast_convert_type(v, jnp.int32)
    keyv = jnp.where(bb < 0, bb ^ _M, bb)
    return keyv ^ _M


def _unskey(sk):
    return jnp.where(sk < 0, sk, sk ^ _M)


@functools.lru_cache(maxsize=None)
def _make_sc_topk():
    mesh = plsc.VectorSubcoreMesh(core_axis_name="c", subcore_axis_name="s")
    n_gchunk = 2 * K // 128
    kbuf = K + CAP + 96

    @functools.partial(
        pl.kernel,
        mesh=mesh,
        compiler_params=pltpu.CompilerParams(needs_layout_passes=False),
        out_type=(
            jax.ShapeDtypeStruct((B, K), jnp.float32),
            jax.ShapeDtypeStruct((B, 2 * K), jnp.float32),
        ),
        scratch_types=[
            pltpu.VMEM((N,), jnp.float32),
            pltpu.VMEM((kbuf,), jnp.int32),
            pltpu.VMEM((kbuf,), jnp.int32),
            pltpu.VMEM((kbuf,), jnp.int32),
            pltpu.VMEM((kbuf,), jnp.int32),
            pltpu.VMEM((2064,), jnp.int32),
            pltpu.VMEM((2064,), jnp.int32),
            pltpu.VMEM((16,), jnp.int32),
            pltpu.VMEM((n_gchunk, 128), jnp.int32),
            pltpu.VMEM((2 * K,), jnp.float32),
            pltpu.VMEM((K,), jnp.float32),
            pltpu.SemaphoreType.DMA,
        ],
    )
    def topk_k(
        scores_hbm, kflat_hbm, oscore_hbm, okpts_hbm,
        sv, kb0, ib0, kb1, ib1, hist, base, spill, gidx, gbuf, sbuf, sem,
    ):
        wid = lax.axis_index("s") * _SC_CORES + lax.axis_index("c")

        @pl.when(wid < B)
        def _():
            pltpu.sync_copy(scores_hbm.at[wid], sv)
            lanes = lax.broadcasted_iota(jnp.int32, (16,), 0)
            zeros16 = jnp.zeros((16,), jnp.int32)

            def run_level(sh, msk, nbins, lo, hi, below, first):
                for j in range(2064 // 16):
                    hist[pl.ds(j * 16, 16)] = zeros16

                ones16 = jnp.full((16,), 1, jnp.int32)

                @plsc.parallel_loop(0, _NV // 8)
                def _hist_loop(i):
                    for u in range(8):
                        v = sv[pl.ds((i * 8 + u) * 16, 16)]
                        sk = _skey(v)
                        d = (sk >> sh) & msk
                        if not first:
                            ssk = sk ^ _SIGN
                            cand = (ssk >= lo) & (ssk <= hi)
                            d = jnp.where(cand, d, nbins)
                        plsc.addupdate_scatter(hist, [d], ones16)

                local_rank = K - below

                def pbody(j, carry):
                    run, nlt = carry
                    hv = hist[pl.ds(j * 16, 16)]
                    c = plsc.cumsum(hv) + run
                    base[pl.ds(j * 16, 16)] = c - hv
                    nlt = nlt + plsc.all_reduce_population_count(
                        c < local_rank
                    )[0]
                    return run + jnp.sum(hv), nlt

                _, bstar = lax.fori_loop(
                    0, nbins // 16, pbody, (jnp.int32(0), jnp.int32(0))
                )
                bvec = jnp.broadcast_to(bstar, (16,))
                below_here = plsc.load_gather(base, [bvec])[0]
                cnt = plsc.load_gather(hist, [bvec])[0]
                return bstar, below + below_here, cnt

            b0, below, cnt = run_level(21, 0x7FF, 2048, 0, 0, jnp.int32(0), True)
            ub = b0 << 21
            span = 1 << 21

            def lvl(sh, msk, nbins, ub, span, below, cnt):
                lo = ub ^ _SIGN
                hi = (ub + span - 1) ^ _SIGN
                bn, below2, cnt2 = run_level(sh, msk, nbins, lo, hi, below, False)
                return ub | (bn << sh), 1 << sh, below2, cnt2

            st = (ub, span, below, cnt)

            def refine1(st):
                return lvl(10, 0x7FF, 2048, st[0], st[1], st[2], st[3])

            def refine2(st):
                return lvl(0, 0x3FF, 1024, st[0], st[1], st[2], st[3])

            st = lax.cond(st[3] > CAP, refine1, lambda s: s, st)
            st = lax.cond(st[3] > CAP, refine2, lambda s: s, st)
            ub, span, below, cnt = st

            lo_cut = jnp.broadcast_to(ub ^ _SIGN, (16,))
            hi_cut = jnp.broadcast_to((ub + span - 1) ^ _SIGN, (16,))
            cap_total = jnp.where(cnt <= CAP, below + cnt, jnp.int32(K))

            @plsc.parallel_loop(0, _NV // 4, carry=(jnp.int32(0), below))
            def cbody(i, carry):
                w_off, c_off = carry
                for u in range(4):
                    ii = i * 4 + u
                    v = sv[pl.ds(ii * 16, 16)]
                    sk = _skey(v)
                    ssk = sk ^ _SIGN
                    iv = lanes + ii * 16
                    win = ssk < lo_cut
                    cand = (ssk >= lo_cut) & (ssk <= hi_cut)
                    nw = plsc.all_reduce_population_count(win)[0]
                    plsc.store_compressed(kb0.at[pl.ds(w_off, 16)], sk, mask=win)
                    plsc.store_compressed(ib0.at[pl.ds(w_off, 16)], iv, mask=win)
                    nk = plsc.all_reduce_population_count(cand)[0]
                    plsc.store_compressed(kb0.at[pl.ds(c_off, 16)], sk, mask=cand)
                    plsc.store_compressed(ib0.at[pl.ds(c_off, 16)], iv, mask=cand)
                    w_off = w_off + nw
                    c_off = jnp.minimum(c_off + nk, cap_total)
                return w_off, c_off

            for u in range(4):
                kb0[pl.ds(cap_total + u * 16, 16)] = jnp.full((16,), -1, jnp.int32)
                ib0[pl.ds(cap_total + u * 16, 16)] = jnp.full((16,), N, jnp.int32)
            nv_sort = (cap_total + 63) >> 6

            bufs = [(kb0, ib0, kb1, ib1), (kb1, ib1, kb0, ib0)]
            for p in range(4):
                src_k, src_i, dst_k, dst_i = bufs[p % 2]
                shift = 8 * p
                for j in range(16):
                    hist[pl.ds(j * 16, 16)] = zeros16

                ones16 = jnp.full((16,), 1, jnp.int32)

                @plsc.parallel_loop(0, nv_sort)
                def _sort_hist(i, src_k=src_k, shift=shift):
                    for u in range(4):
                        kv = src_k[pl.ds((i * 4 + u) * 16, 16)]
                        d = (kv >> shift) & 255
                        plsc.addupdate_scatter(hist, [d], ones16)

                def pbody2(j, run):
                    hv = hist[pl.ds(j * 16, 16)]
                    c = plsc.cumsum(hv)
                    base[pl.ds(j * 16, 16)] = (run + c) - hv
                    return run + jnp.sum(hv)

                lax.fori_loop(0, 16, pbody2, jnp.int32(0))

                def mbody2(i, carry, src_k=src_k, src_i=src_i,
                           dst_k=dst_k, dst_i=dst_i, shift=shift):
                    for u in range(4):
                        ii = i * 4 + u
                        kv = src_k[pl.ds(ii * 16, 16)]
                        ix = src_i[pl.ds(ii * 16, 16)]
                        d = (kv >> shift) & 255
                        a, last = plsc.scan_count(d)
                        bs = plsc.load_gather(base, [d])
                        pos = bs + a - 1
                        plsc.store_scatter(dst_k, [pos], kv)
                        plsc.store_scatter(dst_i, [pos], ix)
                        plsc.addupdate_scatter(base, [d], a, mask=last)
                    return carry

                lax.fori_loop(0, nv_sort, mbody2, jnp.int32(0))


            rbase2 = wid * (2 * N)

            @plsc.parallel_loop(0, K // 16 // 4)
            def _emit_loop(i):
                for u in range(4):
                    ii = i * 4 + u
                    sk = kb0[pl.ds(ii * 16, 16)]
                    sbuf[pl.ds(ii * 16, 16)] = lax.bitcast_convert_type(
                        _unskey(sk), jnp.float32
                    )
                    ix = ib0[pl.ds(ii * 16, 16)]
                    e0 = rbase2 + ix * 2
                    p0 = (lanes + ii * 16) * 2
                    plsc.store_scatter(gidx, [p0 >> 7, p0 & 127], e0)
                    plsc.store_scatter(
                        gidx, [(p0 + 1) >> 7, (p0 + 1) & 127], e0 + 1
                    )
            pltpu.sync_copy(sbuf, oscore_hbm.at[wid])

            copies = [
                pltpu.async_copy(
                    kflat_hbm.at[gidx.at[j]], gbuf.at[pl.ds(j * 128, 128)], sem
                )
                for j in range(n_gchunk)
            ]
            for cpy in copies:
                cpy.wait()
            pltpu.sync_copy(gbuf, okpts_hbm.at[wid])

    return topk_k


def kernel(keypoints, scores):
    top_scores, kpts_flat = _make_sc_topk()(scores, keypoints.reshape(2 * B * N))
    return (kpts_flat.reshape(B, K, 2), top_scores)


# speedup vs baseline: 1.2399x; 1.0018x over previous
import functools

import jax
import jax.numpy as jnp
from jax import lax
from jax.experimental import pallas as pl
from jax.experimental.pallas import tpu as pltpu
from jax.experimental.pallas import tpu_sc as plsc

B = 8
N = 32768
K = 2048
CAP = 6144

_SC_CORES = 2
_SC_SUBCORES = 16
_M = 0x7FFFFFFF
_SIGN = -2147483648
_NV = N // 16


def _skey(v):
    bb = lax.bitcast_convert_type(v, jnp.int32)
    keyv = jnp.where(bb < 0, bb ^ _M, bb)
    return keyv ^ _M


def _unskey(sk):
    return jnp.where(sk < 0, sk, sk ^ _M)


@functools.lru_cache(maxsize=None)
def _make_sc_topk():
    mesh = plsc.VectorSubcoreMesh(core_axis_name="c", subcore_axis_name="s")
    n_gchunk = 2 * K // 128
    kbuf = K + CAP + 96

    @functools.partial(
        pl.kernel,
        mesh=mesh,
        compiler_params=pltpu.CompilerParams(needs_layout_passes=False),
        out_type=(
            jax.ShapeDtypeStruct((B, K), jnp.float32),
            jax.ShapeDtypeStruct((B, 2 * K), jnp.float32),
        ),
        scratch_types=[
            pltpu.VMEM((N,), jnp.float32),
            pltpu.VMEM((kbuf,), jnp.int32),
            pltpu.VMEM((kbuf,), jnp.int32),
            pltpu.VMEM((kbuf,), jnp.int32),
            pltpu.VMEM((kbuf,), jnp.int32),
            pltpu.VMEM((2064,), jnp.int32),
            pltpu.VMEM((2064,), jnp.int32),
            pltpu.VMEM((16,), jnp.int32),
            pltpu.VMEM((n_gchunk, 128), jnp.int32),
            pltpu.VMEM((2 * K,), jnp.float32),
            pltpu.VMEM((K,), jnp.float32),
            pltpu.SemaphoreType.DMA,
        ],
    )
    def topk_k(
        scores_hbm, kflat_hbm, oscore_hbm, okpts_hbm,
        sv, kb0, ib0, kb1, ib1, hist, base, spill, gidx, gbuf, sbuf, sem,
    ):
        wid = lax.axis_index("s") * _SC_CORES + lax.axis_index("c")

        @pl.when(wid < B)
        def _():
            pltpu.sync_copy(scores_hbm.at[wid], sv)
            lanes = lax.broadcasted_iota(jnp.int32, (16,), 0)
            zeros16 = jnp.zeros((16,), jnp.int32)

            def run_level(sh, msk, nbins, lo, hi, below, first):
                for j in range(2064 // 16):
                    hist[pl.ds(j * 16, 16)] = zeros16

                ones16 = jnp.full((16,), 1, jnp.int32)

                @plsc.parallel_loop(0, _NV // 8)
                def _hist_loop(i):
                    for u in range(8):
                        v = sv[pl.ds((i * 8 + u) * 16, 16)]
                        sk = _skey(v)
                        d = (sk >> sh) & msk
                        if not first:
                            ssk = sk ^ _SIGN
                            cand = (ssk >= lo) & (ssk <= hi)
                            d = jnp.where(cand, d, nbins)
                        plsc.addupdate_scatter(hist, [d], ones16)

                local_rank = K - below

                def pbody(j, carry):
                    run, nlt = carry
                    hv = hist[pl.ds(j * 16, 16)]
                    c = plsc.cumsum(hv) + run
                    base[pl.ds(j * 16, 16)] = c - hv
                    nlt = nlt + plsc.all_reduce_population_count(
                        c < local_rank
                    )[0]
                    return run + jnp.sum(hv), nlt

                _, bstar = lax.fori_loop(
                    0, nbins // 16, pbody, (jnp.int32(0), jnp.int32(0))
                )
                bvec = jnp.broadcast_to(bstar, (16,))
                below_here = plsc.load_gather(base, [bvec])[0]
                cnt = plsc.load_gather(hist, [bvec])[0]
                return bstar, below + below_here, cnt

            b0, below, cnt = run_level(21, 0x7FF, 2048, 0, 0, jnp.int32(0), True)
            ub = b0 << 21
            span = 1 << 21

            def lvl(sh, msk, nbins, ub, span, below, cnt):
                lo = ub ^ _SIGN
                hi = (ub + span - 1) ^ _SIGN
                bn, below2, cnt2 = run_level(sh, msk, nbins, lo, hi, below, False)
                return ub | (bn << sh), 1 << sh, below2, cnt2

            st = (ub, span, below, cnt)

            def refine1(st):
                return lvl(10, 0x7FF, 2048, st[0], st[1], st[2], st[3])

            def refine2(st):
                return lvl(0, 0x3FF, 1024, st[0], st[1], st[2], st[3])

            st = lax.cond(st[3] > CAP, refine1, lambda s: s, st)
            st = lax.cond(st[3] > CAP, refine2, lambda s: s, st)
            ub, span, below, cnt = st

            lo_cut = jnp.broadcast_to(ub ^ _SIGN, (16,))
            hi_cut = jnp.broadcast_to((ub + span - 1) ^ _SIGN, (16,))
            cap_total = jnp.where(cnt <= CAP, below + cnt, jnp.int32(K))

            cap_vec = jnp.broadcast_to(cap_total, (16,))
            trash = jnp.full((16,), kbuf - 16, jnp.int32)
            w0 = jnp.zeros((16,), jnp.int32)
            c0 = jnp.broadcast_to(below, (16,))

            @plsc.parallel_loop(0, _NV // 4, carry=(w0, c0))
            def cbody(i, carry):
                w_off, c_off = carry
                for u in range(4):
                    ii = i * 4 + u
                    v = sv[pl.ds(ii * 16, 16)]
                    sk = _skey(v)
                    ssk = sk ^ _SIGN
                    iv = lanes + ii * 16
                    win = ssk < lo_cut
                    cand = (ssk >= lo_cut) & (ssk <= hi_cut)
                    cw = plsc.cumsum(win.astype(jnp.int32))
                    dw = jnp.where(win, w_off + cw - 1, trash)
                    plsc.store_scatter(kb0, [dw], sk)
                    plsc.store_scatter(ib0, [dw], iv)
                    cc = plsc.cumsum(cand.astype(jnp.int32))
                    dc = jnp.where(
                        cand, jnp.minimum(c_off + cc - 1, cap_vec), trash
                    )
                    plsc.store_scatter(kb0, [dc], sk)
                    plsc.store_scatter(ib0, [dc], iv)
                    nw = plsc.all_reduce_population_count(win)
                    nc = plsc.all_reduce_population_count(cand)
                    w_off = w_off + nw
                    c_off = jnp.minimum(c_off + nc, cap_vec)
                return w_off, c_off

            for u in range(4):
                kb0[pl.ds(cap_total + u * 16, 16)] = jnp.full((16,), -1, jnp.int32)
                ib0[pl.ds(cap_total + u * 16, 16)] = jnp.full((16,), N, jnp.int32)
            nv_sort = (cap_total + 63) >> 6

            bufs = [(kb0, ib0, kb1, ib1), (kb1, ib1, kb0, ib0)]
            for p in range(4):
                src_k, src_i, dst_k, dst_i = bufs[p % 2]
                shift = 8 * p
                for j in range(16):
                    hist[pl.ds(j * 16, 16)] = zeros16

                ones16 = jnp.full((16,), 1, jnp.int32)

                @plsc.parallel_loop(0, nv_sort)
                def _sort_hist(i, src_k=src_k, shift=shift):
                    for u in range(4):
                        kv = src_k[pl.ds((i * 4 + u) * 16, 16)]
                        d = (kv >> shift) & 255
                        plsc.addupdate_scatter(hist, [d], ones16)

                def pbody2(j, run):
                    hv = hist[pl.ds(j * 16, 16)]
                    c = plsc.cumsum(hv)
                    base[pl.ds(j * 16, 16)] = (run + c) - hv
                    return run + jnp.sum(hv)

                lax.fori_loop(0, 16, pbody2, jnp.int32(0))

                def mbody2(i, carry, src_k=src_k, src_i=src_i,
                           dst_k=dst_k, dst_i=dst_i, shift=shift):
                    for u in range(4):
                        ii = i * 4 + u
                        kv = src_k[pl.ds(ii * 16, 16)]
                        ix = src_i[pl.ds(ii * 16, 16)]
                        d = (kv >> shift) & 255
                        a, last = plsc.scan_count(d)
                        bs = plsc.load_gather(base, [d])
                        pos = bs + a - 1
                        plsc.store_scatter(dst_k, [pos], kv)
                        plsc.store_scatter(dst_i, [pos], ix)
                        plsc.addupdate_scatter(base, [d], a, mask=last)
                    return carry

                lax.fori_loop(0, nv_sort, mbody2, jnp.int32(0))


            rbase2 = wid * (2 * N)

            @plsc.parallel_loop(0, K // 16 // 4)
            def _emit_loop(i):
                for u in range(4):
                    ii = i * 4 + u
                    sk = kb0[pl.ds(ii * 16, 16)]
                    sbuf[pl.ds(ii * 16, 16)] = lax.bitcast_convert_type(
                        _unskey(sk), jnp.float32
                    )
                    ix = ib0[pl.ds(ii * 16, 16)]
                    e0 = rbase2 + ix * 2
                    p0 = (lanes + ii * 16) * 2
                    plsc.store_scatter(gidx, [p0 >> 7, p0 & 127], e0)
                    plsc.store_scatter(
                        gidx, [(p0 + 1) >> 7, (p0 + 1) & 127], e0 + 1
                    )
            pltpu.sync_copy(sbuf, oscore_hbm.at[wid])

            copies = [
                pltpu.async_copy(
                    kflat_hbm.at[gidx.at[j]], gbuf.at[pl.ds(j * 128, 128)], sem
                )
                for j in range(n_gchunk)
            ]
            for cpy in copies:
                cpy.wait()
            pltpu.sync_copy(gbuf, okpts_hbm.at[wid])

    return topk_k


def kernel(keypoints, scores):
    top_scores, kpts_flat = _make_sc_topk()(scores, keypoints.reshape(2 * B * N))
    return (kpts_flat.reshape(B, K, 2), top_scores)


# speedup vs baseline: 3.4756x; 2.8032x over previous
import functools

import jax
import jax.numpy as jnp
from jax import lax
from jax.experimental import pallas as pl
from jax.experimental.pallas import tpu as pltpu
from jax.experimental.pallas import tpu_sc as plsc

B = 8
N = 32768
K = 2048
CAP = 6144

_SC_CORES = 2
_SC_SUBCORES = 16
_M = 0x7FFFFFFF
_SIGN = -2147483648
_NV = N // 16


def _skey(v):
    bb = lax.bitcast_convert_type(v, jnp.int32)
    keyv = jnp.where(bb < 0, bb ^ _M, bb)
    return keyv ^ _M


def _unskey(sk):
    return jnp.where(sk < 0, sk, sk ^ _M)


@functools.lru_cache(maxsize=None)
def _make_sc_topk():
    mesh = plsc.VectorSubcoreMesh(core_axis_name="c", subcore_axis_name="s")
    n_gchunk = K // 128
    kbuf = K + CAP + 96

    @functools.partial(
        pl.kernel,
        mesh=mesh,
        compiler_params=pltpu.CompilerParams(needs_layout_passes=False),
        out_type=(
            jax.ShapeDtypeStruct((B, K), jnp.float32),
            jax.ShapeDtypeStruct((B, 2 * K), jnp.float32),
        ),
        scratch_types=[
            pltpu.VMEM((N,), jnp.float32),
            pltpu.VMEM((kbuf,), jnp.int32),
            pltpu.VMEM((kbuf,), jnp.int32),
            pltpu.VMEM((kbuf,), jnp.int32),
            pltpu.VMEM((kbuf,), jnp.int32),
            pltpu.VMEM((2064,), jnp.int32),
            pltpu.VMEM((2064,), jnp.int32),
            pltpu.VMEM((16,), jnp.int32),
            pltpu.VMEM((n_gchunk, 128), jnp.int32),
            pltpu.VMEM((K,), jnp.float32),
            pltpu.VMEM((K,), jnp.float32),
            pltpu.VMEM((2 * K,), jnp.float32),
            pltpu.VMEM((K,), jnp.float32),
            pltpu.SemaphoreType.DMA,
        ],
    )
    def topk_k(
        scores_hbm, kx_hbm, ky_hbm, oscore_hbm, okpts_hbm,
        sv, kb0, ib0, kb1, ib1, hist, base, spill, gidx, gxb, gyb, gbuf, sbuf,
        sem,
    ):
        wid = lax.axis_index("s") * _SC_CORES + lax.axis_index("c")

        @pl.when(wid < B)
        def _():
            pltpu.sync_copy(scores_hbm.at[wid], sv)
            lanes = lax.broadcasted_iota(jnp.int32, (16,), 0)
            zeros16 = jnp.zeros((16,), jnp.int32)

            def run_level(sh, msk, nbins, lo, hi, below, first):
                for j in range(2064 // 16):
                    hist[pl.ds(j * 16, 16)] = zeros16

                ones16 = jnp.full((16,), 1, jnp.int32)

                @plsc.parallel_loop(0, _NV // 8)
                def _hist_loop(i):
                    for u in range(8):
                        v = sv[pl.ds((i * 8 + u) * 16, 16)]
                        sk = _skey(v)
                        d = (sk >> sh) & msk
                        if not first:
                            ssk = sk ^ _SIGN
                            cand = (ssk >= lo) & (ssk <= hi)
                            d = jnp.where(cand, d, nbins)
                        plsc.addupdate_scatter(hist, [d], ones16)

                local_rank = K - below

                def pbody(j, carry):
                    run, nlt = carry
                    hv = hist[pl.ds(j * 16, 16)]
                    c = plsc.cumsum(hv) + run
                    base[pl.ds(j * 16, 16)] = c - hv
                    nlt = nlt + plsc.all_reduce_population_count(
                        c < local_rank
                    )[0]
                    return run + jnp.sum(hv), nlt

                _, bstar = lax.fori_loop(
                    0, nbins // 16, pbody, (jnp.int32(0), jnp.int32(0))
                )
                bvec = jnp.broadcast_to(bstar, (16,))
                below_here = plsc.load_gather(base, [bvec])[0]
                cnt = plsc.load_gather(hist, [bvec])[0]
                return bstar, below + below_here, cnt

            b0, below, cnt = run_level(21, 0x7FF, 2048, 0, 0, jnp.int32(0), True)
            ub = b0 << 21
            span = 1 << 21

            def lvl(sh, msk, nbins, ub, span, below, cnt):
                lo = ub ^ _SIGN
                hi = (ub + span - 1) ^ _SIGN
                bn, below2, cnt2 = run_level(sh, msk, nbins, lo, hi, below, False)
                return ub | (bn << sh), 1 << sh, below2, cnt2

            st = (ub, span, below, cnt)

            def refine1(st):
                return lvl(10, 0x7FF, 2048, st[0], st[1], st[2], st[3])

            def refine2(st):
                return lvl(0, 0x3FF, 1024, st[0], st[1], st[2], st[3])

            st = lax.cond(st[3] > CAP, refine1, lambda s: s, st)
            st = lax.cond(st[3] > CAP, refine2, lambda s: s, st)
            ub, span, below, cnt = st

            lo_cut = jnp.broadcast_to(ub ^ _SIGN, (16,))
            hi_cut = jnp.broadcast_to((ub + span - 1) ^ _SIGN, (16,))
            cap_total = jnp.where(cnt <= CAP, below + cnt, jnp.int32(K))

            cap_vec = jnp.broadcast_to(cap_total, (16,))
            trash = jnp.full((16,), kbuf - 16, jnp.int32)
            w0 = jnp.zeros((16,), jnp.int32)
            c0 = jnp.broadcast_to(below, (16,))

            @plsc.parallel_loop(0, _NV // 4, carry=(w0, c0))
            def cbody(i, carry):
                w_off, c_off = carry
                for u in range(4):
                    ii = i * 4 + u
                    v = sv[pl.ds(ii * 16, 16)]
                    sk = _skey(v)
                    ssk = sk ^ _SIGN
                    iv = lanes + ii * 16
                    win = ssk < lo_cut
                    cand = (ssk >= lo_cut) & (ssk <= hi_cut)
                    cw = plsc.cumsum(win.astype(jnp.int32))
                    dw = jnp.where(win, w_off + cw - 1, trash)
                    plsc.store_scatter(kb0, [dw], sk)
                    plsc.store_scatter(ib0, [dw], iv)
                    cc = plsc.cumsum(cand.astype(jnp.int32))
                    dc = jnp.where(
                        cand, jnp.minimum(c_off + cc - 1, cap_vec), trash
                    )
                    plsc.store_scatter(kb0, [dc], sk)
                    plsc.store_scatter(ib0, [dc], iv)
                    nw = plsc.all_reduce_population_count(win)
                    nc = plsc.all_reduce_population_count(cand)
                    w_off = w_off + nw
                    c_off = jnp.minimum(c_off + nc, cap_vec)
                return w_off, c_off

            for u in range(4):
                kb0[pl.ds(cap_total + u * 16, 16)] = jnp.full((16,), -1, jnp.int32)
                ib0[pl.ds(cap_total + u * 16, 16)] = jnp.full((16,), N, jnp.int32)
            nv_sort = (cap_total + 63) >> 6

            bufs = [(kb0, ib0, kb1, ib1), (kb1, ib1, kb0, ib0)]
            for p in range(4):
                src_k, src_i, dst_k, dst_i = bufs[p % 2]
                shift = 8 * p
                for j in range(16):
                    hist[pl.ds(j * 16, 16)] = zeros16

                ones16 = jnp.full((16,), 1, jnp.int32)

                @plsc.parallel_loop(0, nv_sort)
                def _sort_hist(i, src_k=src_k, shift=shift):
                    for u in range(4):
                        kv = src_k[pl.ds((i * 4 + u) * 16, 16)]
                        d = (kv >> shift) & 255
                        plsc.addupdate_scatter(hist, [d], ones16)

                def pbody2(j, run):
                    hv = hist[pl.ds(j * 16, 16)]
                    c = plsc.cumsum(hv)
                    base[pl.ds(j * 16, 16)] = (run + c) - hv
                    return run + jnp.sum(hv)

                lax.fori_loop(0, 16, pbody2, jnp.int32(0))

                def mbody2(i, carry, src_k=src_k, src_i=src_i,
                           dst_k=dst_k, dst_i=dst_i, shift=shift):
                    for u in range(4):
                        ii = i * 4 + u
                        kv = src_k[pl.ds(ii * 16, 16)]
                        ix = src_i[pl.ds(ii * 16, 16)]
                        d = (kv >> shift) & 255
                        a, last = plsc.scan_count(d)
                        bs = plsc.load_gather(base, [d])
                        pos = bs + a - 1
                        plsc.store_scatter(dst_k, [pos], kv)
                        plsc.store_scatter(dst_i, [pos], ix)
                        plsc.addupdate_scatter(base, [d], a, mask=last)
                    return carry

                lax.fori_loop(0, nv_sort, mbody2, jnp.int32(0))


            rbase = wid * N

            @plsc.parallel_loop(0, K // 16 // 4)
            def _emit_loop(i):
                for u in range(4):
                    ii = i * 4 + u
                    sk = kb0[pl.ds(ii * 16, 16)]
                    sbuf[pl.ds(ii * 16, 16)] = lax.bitcast_convert_type(
                        _unskey(sk), jnp.float32
                    )
                    ix = ib0[pl.ds(ii * 16, 16)]
                    p0 = lanes + ii * 16
                    plsc.store_scatter(gidx, [p0 >> 7, p0 & 127], rbase + ix)
            pltpu.sync_copy(sbuf, oscore_hbm.at[wid])

            copies = [
                pltpu.async_copy(
                    kx_hbm.at[gidx.at[j]], gxb.at[pl.ds(j * 128, 128)], sem
                )
                for j in range(n_gchunk)
            ] + [
                pltpu.async_copy(
                    ky_hbm.at[gidx.at[j]], gyb.at[pl.ds(j * 128, 128)], sem
                )
                for j in range(n_gchunk)
            ]
            for cpy in copies:
                cpy.wait()

            @plsc.parallel_loop(0, K // 16 // 4)
            def _ilv_loop(i):
                for u in range(4):
                    ii = i * 4 + u
                    xv = gxb[pl.ds(ii * 16, 16)]
                    yv = gyb[pl.ds(ii * 16, 16)]
                    p0 = (lanes + ii * 16) * 2
                    plsc.store_scatter(gbuf, [p0], xv)
                    plsc.store_scatter(gbuf, [p0 + 1], yv)

            pltpu.sync_copy(gbuf, okpts_hbm.at[wid])

    return topk_k


def kernel(keypoints, scores):
    kx = keypoints[:, :, 0].reshape(B * N)
    ky = keypoints[:, :, 1].reshape(B * N)
    top_scores, kpts_flat = _make_sc_topk()(scores, kx, ky)
    return (kpts_flat.reshape(B, K, 2), top_scores)
